# Initial kernel scaffold; baseline (speedup 1.0000x reference)
import jax
import jax.numpy as jnp
from jax.experimental import pallas as pl


def kernel(x, edge_index):
    raise NotImplementedError("write your pallas kernel here")


# speedup vs baseline: 4.9808x; 4.9808x over previous
import functools

import jax
import jax.numpy as jnp
from jax import lax
from jax.experimental import pallas as pl
from jax.experimental.pallas import tpu as pltpu
from jax.experimental.pallas import tpu_sc as plsc

N_NODES = 10000
N_EDGES = 320000
D_FEAT = 128

NC = 2
NS = 16
NW = NC * NS
CHUNK = 128
E_PAD = ((N_EDGES + NW * CHUNK - 1) // (NW * CHUNK)) * (NW * CHUNK)
CH = E_PAD // (NW * CHUNK)
RSTRIDE = 624
ZROWS = 648
WROWS = 640
ACC_ROWS = RSTRIDE * (NS - 1) + ZROWS


def _sc_partials(x, src_w, dst_w, zeros):
    mesh = plsc.VectorSubcoreMesh(core_axis_name="c", subcore_axis_name="s")

    @functools.partial(
        pl.kernel,
        out_type=jax.ShapeDtypeStruct((NC, N_NODES, D_FEAT), jnp.float32),
        mesh=mesh,
        scratch_types=[
            pltpu.VMEM((CH, CHUNK), jnp.int32),
            pltpu.VMEM((CH, CHUNK), jnp.int32),
            pltpu.VMEM((CHUNK, D_FEAT), jnp.float32),
            pltpu.VMEM_SHARED((ACC_ROWS, D_FEAT), jnp.float32),
            pltpu.SemaphoreType.DMA,
        ],
    )
    def body(x_hbm, src_hbm, dst_hbm, z_hbm, out_hbm, src_v, dst_v, rows_v,
             acc, sem):
        c = lax.axis_index("c")
        s = lax.axis_index("s")
        wid = s * NC + c

        pltpu.sync_copy(z_hbm, acc.at[pl.ds(s * RSTRIDE, ZROWS)])
        plsc.subcore_barrier()

        pltpu.sync_copy(src_hbm.at[wid], src_v)
        pltpu.sync_copy(dst_hbm.at[wid], dst_v)

        def chunk_step(j, carry):
            pltpu.async_copy(x_hbm.at[src_v.at[j]], rows_v, sem).wait()
            pltpu.sync_copy(rows_v, acc.at[dst_v.at[j]], add=True)
            return carry

        lax.fori_loop(0, CH, chunk_step, 0)
        plsc.subcore_barrier()

        pltpu.sync_copy(acc.at[pl.ds(s * RSTRIDE, WROWS)],
                        out_hbm.at[c, pl.ds(s * RSTRIDE, WROWS)])

    return body(x, src_w, dst_w, zeros)


def _combine(p):
    def add_body(a_ref, b_ref, o_ref):
        o_ref[...] = a_ref[0] + b_ref[0]

    grid = 10
    blk = N_NODES // grid
    return pl.pallas_call(
        add_body,
        grid=(grid,),
        in_specs=[
            pl.BlockSpec((1, blk, D_FEAT), lambda i: (0, i, 0)),
            pl.BlockSpec((1, blk, D_FEAT), lambda i: (1, i, 0)),
        ],
        out_specs=pl.BlockSpec((blk, D_FEAT), lambda i: (i, 0)),
        out_shape=jax.ShapeDtypeStruct((N_NODES, D_FEAT), jnp.float32),
    )(p, p)


def kernel(x, edge_index):
    src = edge_index[0].astype(jnp.int32)
    dst = edge_index[1].astype(jnp.int32)
    pad = E_PAD - N_EDGES
    src_w = jnp.concatenate([src, jnp.zeros((pad,), jnp.int32)])
    dst_w = jnp.concatenate([dst, jnp.full((pad,), N_NODES, jnp.int32)])
    src_w = src_w.reshape(NW, CH, CHUNK)
    dst_w = dst_w.reshape(NW, CH, CHUNK)
    zeros = jnp.zeros((ZROWS, D_FEAT), jnp.float32)
    partials = _sc_partials(x, src_w, dst_w, zeros)
    return _combine(partials)
